# baseline (speedup 1.0000x reference)
import jax
import jax.numpy as jnp
from jax.experimental import pallas as pl
from jax.experimental.pallas import tpu as pltpu

_TOP_K = 8


def _moe_kernel(x_ref, wgate_ref, wsg_ref,
                wg_hbm, wu_hbm, wd_hbm, wgs_hbm, wus_hbm, wds_hbm,
                out_ref, aux_ref,
                w_ref, shg_ref, shu_ref, shd_ref, cnt_ref, sems):
    T, E = x_ref.shape[0], wgate_ref.shape[1]
    lane = jax.lax.broadcasted_iota(jnp.int32, (T, E), 1)

    cp_g = pltpu.make_async_copy(wgs_hbm, shg_ref, sems.at[0])
    cp_u = pltpu.make_async_copy(wus_hbm, shu_ref, sems.at[1])
    cp_d = pltpu.make_async_copy(wds_hbm, shd_ref, sems.at[2])
    cp_g.start()
    cp_u.start()
    cp_d.start()

    xbf = x_ref[...].astype(jnp.bfloat16)
    out_ref[...] = jnp.zeros(out_ref.shape, jnp.float32)
    cnt_ref[0] = 0

    def body(wg_blk, wu_blk, wd_blk):
        e = cnt_ref[0]
        cnt_ref[0] = e + 1

        @pl.when(e == 0)
        def _router():
            logits = jnp.dot(xbf, wgate_ref[...].astype(jnp.bfloat16),
                             preferred_element_type=jnp.float32)
            m = jnp.max(logits, axis=1, keepdims=True)
            ex = jnp.exp(logits - m)
            probs = ex / jnp.sum(ex, axis=1, keepdims=True)
            rem = probs
            sel = jnp.zeros((T, E), jnp.float32)
            for _ in range(_TOP_K):
                cur = jnp.max(rem, axis=1, keepdims=True)
                is_max = rem == cur
                first = jnp.min(jnp.where(is_max, lane, E),
                                axis=1, keepdims=True)
                pick = lane == first
                sel = jnp.where(pick, probs, sel)
                rem = jnp.where(pick, -1.0, rem)
            w_ref[...] = sel
            usage = jnp.mean(probs, axis=0, keepdims=True)
            aux_ref[...] = jnp.mean((usage - 1.0 / E) ** 2, keepdims=True)

        wg = wg_blk[0].astype(jnp.bfloat16)
        wu = wu_blk[0].astype(jnp.bfloat16)
        wd = wd_blk[0].astype(jnp.bfloat16)
        ehg = jnp.dot(xbf, wg, preferred_element_type=jnp.float32)
        ehu = jnp.dot(xbf, wu, preferred_element_type=jnp.float32)
        ehh = (ehg * jax.nn.sigmoid(ehg) * ehu).astype(jnp.bfloat16)
        eo = jnp.dot(ehh, wd, preferred_element_type=jnp.float32)
        wcol = jnp.sum(jnp.where(lane == e, w_ref[...], 0.0),
                       axis=1, keepdims=True)
        out_ref[...] += wcol * eo

        @pl.when(e == E // 2)
        def _shared():
            cp_g.wait()
            cp_u.wait()
            cp_d.wait()
            hg = jnp.dot(xbf, shg_ref[...].astype(jnp.bfloat16),
                         preferred_element_type=jnp.float32)
            hu = jnp.dot(xbf, shu_ref[...].astype(jnp.bfloat16),
                         preferred_element_type=jnp.float32)
            hh = (hg * jax.nn.sigmoid(hg) * hu).astype(jnp.bfloat16)
            so = jnp.dot(hh, shd_ref[...].astype(jnp.bfloat16),
                         preferred_element_type=jnp.float32)
            gate = jax.nn.sigmoid(
                jnp.sum(xbf.astype(jnp.float32)
                        * wsg_ref[...].astype(jnp.bfloat16)
                        .astype(jnp.float32),
                        axis=1, keepdims=True))
            out_ref[...] += gate * so

    d = x_ref.shape[1]
    F = wg_hbm.shape[2]
    pipeline = pltpu.emit_pipeline(
        body,
        grid=(wg_hbm.shape[0],),
        in_specs=[
            pl.BlockSpec((1, d, F), lambda e: (e, 0, 0)),
            pl.BlockSpec((1, d, F), lambda e: (e, 0, 0)),
            pl.BlockSpec((1, F, d), lambda e: (e, 0, 0)),
        ],
    )
    pipeline(wg_hbm, wu_hbm, wd_hbm)


def kernel(hidden_states, W_gate, Wg, Wu, Wd, Wg_s, Wu_s, Wd_s, W_sg):
    b, s, d = hidden_states.shape
    T = b * s
    E = W_gate.shape[1]
    F = Wg.shape[2]
    x = hidden_states.reshape(T, d)
    out, aux = pl.pallas_call(
        _moe_kernel,
        in_specs=[
            pl.BlockSpec(memory_space=pltpu.VMEM),
            pl.BlockSpec(memory_space=pltpu.VMEM),
            pl.BlockSpec(memory_space=pltpu.VMEM),
            pl.BlockSpec(memory_space=pltpu.MemorySpace.HBM),
            pl.BlockSpec(memory_space=pltpu.MemorySpace.HBM),
            pl.BlockSpec(memory_space=pltpu.MemorySpace.HBM),
            pl.BlockSpec(memory_space=pltpu.MemorySpace.HBM),
            pl.BlockSpec(memory_space=pltpu.MemorySpace.HBM),
            pl.BlockSpec(memory_space=pltpu.MemorySpace.HBM),
        ],
        out_specs=[
            pl.BlockSpec(memory_space=pltpu.VMEM),
            pl.BlockSpec(memory_space=pltpu.VMEM),
        ],
        out_shape=[
            jax.ShapeDtypeStruct((T, d), jnp.float32),
            jax.ShapeDtypeStruct((1, 1), jnp.float32),
        ],
        scratch_shapes=[
            pltpu.VMEM((T, E), jnp.float32),
            pltpu.VMEM((d, F), jnp.float32),
            pltpu.VMEM((d, F), jnp.float32),
            pltpu.VMEM((F, d), jnp.float32),
            pltpu.SMEM((1,), jnp.int32),
            pltpu.SemaphoreType.DMA((3,)),
        ],
    )(x, W_gate, W_sg.reshape(1, d), Wg, Wu, Wd, Wg_s, Wu_s, Wd_s)
    return out.reshape(b, s, d), aux.reshape(())
